# Initial kernel scaffold; baseline (speedup 1.0000x reference)
import functools

import jax
import jax.numpy as jnp
from jax import lax
from jax.experimental import pallas as pl
from jax.experimental.pallas import tpu as pltpu
from jax.experimental.pallas import tpu_sc as plsc

N = 10000
NPAD = 10240
D = 128
E = 320000
EPT = 10112
EPAD = 32 * EPT
NCHUNK = EPT // 128
P = 65536
PPT = P // 32
PCHUNK = PPT // 128

NC = 2
NS = 16
ROWS_PER_TILE = NPAD // NS

_MESH = plsc.VectorSubcoreMesh(core_axis_name="c", subcore_axis_name="s")


@functools.partial(
    pl.kernel,
    out_type=jax.ShapeDtypeStruct((NC, NPAD), jnp.int32),
    mesh=_MESH,
    scratch_types=[
        pltpu.VMEM((128,), jnp.int32),
        pltpu.VMEM((128,), jnp.int32),
        pltpu.VMEM_SHARED((NPAD,), jnp.int32),
    ],
)
def _sc_degree(dst_hbm, zeros_hbm, out_hbm, idx_v, ones_v, hist_sh):
    cid = lax.axis_index("c")
    sid = lax.axis_index("s")
    t = cid * NS + sid

    for k in range(8):
        ones_v[pl.ds(k * 16, 16)] = jnp.ones((16,), jnp.int32)
    pltpu.sync_copy(zeros_hbm.at[pl.ds(sid * ROWS_PER_TILE, ROWS_PER_TILE)],
                    hist_sh.at[pl.ds(sid * ROWS_PER_TILE, ROWS_PER_TILE)])
    plsc.subcore_barrier()

    def body(j, carry):
        base = pl.multiple_of(t * EPT + j * 128, 8)
        pltpu.sync_copy(dst_hbm.at[pl.ds(base, 128)], idx_v)
        pltpu.sync_copy(ones_v, hist_sh.at[idx_v], add=True)
        return carry

    lax.fori_loop(0, NCHUNK, body, 0)
    plsc.subcore_barrier()
    pltpu.sync_copy(hist_sh.at[pl.ds(sid * ROWS_PER_TILE, ROWS_PER_TILE)],
                    out_hbm.at[cid, pl.ds(sid * ROWS_PER_TILE, ROWS_PER_TILE)])


@functools.partial(
    pl.kernel,
    out_type=jax.ShapeDtypeStruct((NC, NPAD, D), jnp.float32),
    mesh=_MESH,
    scratch_types=[
        pltpu.VMEM((128,), jnp.int32),
        pltpu.VMEM((128,), jnp.int32),
        pltpu.VMEM((128, D), jnp.float32),
        pltpu.VMEM_SHARED((NPAD, D), jnp.float32),
        pltpu.SemaphoreType.DMA,
    ],
)
def _sc_scatter(y_hbm, src_hbm, dst_hbm, zeros_hbm, out_hbm,
                src_v, dst_v, rows_v, acc_sh, sem):
    cid = lax.axis_index("c")
    sid = lax.axis_index("s")
    t = cid * NS + sid
    r0 = sid * ROWS_PER_TILE

    pltpu.sync_copy(zeros_hbm.at[pl.ds(r0, ROWS_PER_TILE)],
                    acc_sh.at[pl.ds(r0, ROWS_PER_TILE)])
    plsc.subcore_barrier()

    def body(j, carry):
        base = pl.multiple_of(t * EPT + j * 128, 8)
        pltpu.sync_copy(src_hbm.at[pl.ds(base, 128)], src_v)
        pltpu.sync_copy(dst_hbm.at[pl.ds(base, 128)], dst_v)
        pltpu.async_copy(y_hbm.at[src_v], rows_v, sem).wait()
        pltpu.sync_copy(rows_v, acc_sh.at[dst_v], add=True)
        return carry

    lax.fori_loop(0, NCHUNK, body, 0)
    plsc.subcore_barrier()
    pltpu.sync_copy(acc_sh.at[pl.ds(r0, ROWS_PER_TILE)],
                    out_hbm.at[cid, pl.ds(r0, ROWS_PER_TILE)])


@functools.partial(
    pl.kernel,
    out_type=jax.ShapeDtypeStruct((P,), jnp.float32),
    mesh=_MESH,
    scratch_types=[
        pltpu.VMEM((128,), jnp.int32),
        pltpu.VMEM((128,), jnp.int32),
        pltpu.VMEM((128, D), jnp.float32),
        pltpu.VMEM((128, D), jnp.float32),
        pltpu.VMEM((PPT,), jnp.float32),
        pltpu.SemaphoreType.DMA,
        pltpu.SemaphoreType.DMA,
    ],
)
def _sc_linkpred(h_hbm, sp_hbm, dp_hbm, out_hbm,
                 spv, dpv, buf_a, buf_b, res_v, sem_a, sem_b):
    cid = lax.axis_index("c")
    sid = lax.axis_index("s")
    t = cid * NS + sid
    row_ids = [lax.iota(jnp.int32, 16) + g * 16 for g in range(8)]

    def chunk_body(c, carry):
        base = pl.multiple_of(t * PPT + c * 128, 8)
        pltpu.sync_copy(sp_hbm.at[pl.ds(base, 128)], spv)
        pltpu.sync_copy(dp_hbm.at[pl.ds(base, 128)], dpv)
        cp_a = pltpu.async_copy(h_hbm.at[spv], buf_a, sem_a)
        cp_b = pltpu.async_copy(h_hbm.at[dpv], buf_b, sem_b)
        cp_a.wait()
        cp_b.wait()

        def dot_body(j, accs):
            col = jnp.full((16,), j, jnp.int32)
            new = []
            for g in range(8):
                a = plsc.load_gather(buf_a, [row_ids[g], col])
                b = plsc.load_gather(buf_b, [row_ids[g], col])
                new.append(accs[g] + a * b)
            return tuple(new)

        accs = lax.fori_loop(
            0, D, dot_body,
            tuple(jnp.zeros((16,), jnp.float32) for _ in range(8)))
        for g in range(8):
            sig = 1.0 / (1.0 + jnp.exp(-accs[g]))
            res_v[pl.ds(c * 128 + g * 16, 16)] = sig
        return carry

    lax.fori_loop(0, PCHUNK, chunk_body, 0)
    pltpu.sync_copy(res_v, out_hbm.at[pl.ds(t * PPT, PPT)])


BR = 1280
GRID = NPAD // BR


def _tc1_body(x_ref, w_ref, deg_ref, y_ref, dinv_ref):
    deg = deg_ref[...]
    d = (deg[0] + deg[1] + 1).astype(jnp.float32)
    dinv = lax.rsqrt(d)
    y_ref[...] = jnp.dot(x_ref[...], w_ref[...],
                         preferred_element_type=jnp.float32) * dinv
    dinv_ref[...] = dinv


def _tc2_body(s_ref, y_ref, dinv_ref, b_ref, w_ref, y2_ref):
    s = s_ref[...]
    dinv = dinv_ref[...]
    h = jnp.maximum(dinv * (s[0] + s[1] + y_ref[...]) + b_ref[...], 0.0)
    y2_ref[...] = jnp.dot(h, w_ref[...],
                          preferred_element_type=jnp.float32) * dinv


def _tc3_body(s_ref, y_ref, dinv_ref, b_ref, h_ref):
    s = s_ref[...]
    h_ref[...] = dinv_ref[...] * (s[0] + s[1] + y_ref[...]) + b_ref[...]


_row_spec = pl.BlockSpec((BR, D), lambda i: (i, 0))
_w_spec = pl.BlockSpec((D, D), lambda i: (0, 0))
_dinv_spec = pl.BlockSpec((BR, 1), lambda i: (i, 0))
_s_spec = pl.BlockSpec((NC, BR, D), lambda i: (0, i, 0))
_b_spec = pl.BlockSpec((1, D), lambda i: (0, 0))

_tc1 = pl.pallas_call(
    _tc1_body,
    grid=(GRID,),
    in_specs=[_row_spec, _w_spec, pl.BlockSpec((NC, BR, 1), lambda i: (0, i, 0))],
    out_specs=[_row_spec, _dinv_spec],
    out_shape=[jax.ShapeDtypeStruct((NPAD, D), jnp.float32),
               jax.ShapeDtypeStruct((NPAD, 1), jnp.float32)],
)

_tc2 = pl.pallas_call(
    _tc2_body,
    grid=(GRID,),
    in_specs=[_s_spec, _row_spec, _dinv_spec, _b_spec, _w_spec],
    out_specs=_row_spec,
    out_shape=jax.ShapeDtypeStruct((NPAD, D), jnp.float32),
)

_tc3 = pl.pallas_call(
    _tc3_body,
    grid=(GRID,),
    in_specs=[_s_spec, _row_spec, _dinv_spec, _b_spec],
    out_specs=_row_spec,
    out_shape=jax.ShapeDtypeStruct((NPAD, D), jnp.float32),
)


@jax.jit
def kernel(x, edge_index, edge_pair, W1, b1, W2, b2):
    src = edge_index[0]
    dst = edge_index[1]
    pad = jnp.full((EPAD - E,), NPAD - 1, jnp.int32)
    srcp = jnp.concatenate([src, pad])
    dstp = jnp.concatenate([dst, pad])
    xp = jnp.pad(x, ((0, NPAD - N), (0, 0)))
    zeros_i = jnp.zeros((NPAD,), jnp.int32)
    zeros_f = jnp.zeros((NPAD, D), jnp.float32)
    b1r = b1.reshape(1, D)
    b2r = b2.reshape(1, D)

    degp = _sc_degree(dstp, zeros_i)
    y1, dinv = _tc1(xp, W1, degp.reshape(NC, NPAD, 1))
    s1 = _sc_scatter(y1, srcp, dstp, zeros_f)
    y2 = _tc2(s1, y1, dinv, b1r, W2)
    s2 = _sc_scatter(y2, srcp, dstp, zeros_f)
    h2 = _tc3(s2, y2, dinv, b2r)
    prob = _sc_linkpred(h2, edge_pair[0], edge_pair[1])
    return prob


# speedup vs baseline: 9.9337x; 9.9337x over previous
import functools

import jax
import jax.numpy as jnp
from jax import lax
from jax.experimental import pallas as pl
from jax.experimental.pallas import tpu as pltpu
from jax.experimental.pallas import tpu_sc as plsc

N = 10000
NPAD = 10240
D = 128
E = 320000
EPT = 10112
EPAD = 32 * EPT
NCHUNK = EPT // 128
P = 65536
PPT = P // 32
PCHUNK = PPT // 128

NC = 2
NS = 16
ROWS_PER_TILE = NPAD // NS

_MESH = plsc.VectorSubcoreMesh(core_axis_name="c", subcore_axis_name="s")


@functools.partial(
    pl.kernel,
    out_type=jax.ShapeDtypeStruct((NC, NPAD), jnp.int32),
    mesh=_MESH,
    scratch_types=[
        pltpu.VMEM((128,), jnp.int32),
        pltpu.VMEM((128,), jnp.int32),
        pltpu.VMEM_SHARED((NPAD,), jnp.int32),
    ],
)
def _sc_degree(dst_hbm, zeros_hbm, out_hbm, idx_v, ones_v, hist_sh):
    cid = lax.axis_index("c")
    sid = lax.axis_index("s")
    t = cid * NS + sid

    for k in range(8):
        ones_v[pl.ds(k * 16, 16)] = jnp.ones((16,), jnp.int32)
    pltpu.sync_copy(zeros_hbm.at[pl.ds(sid * ROWS_PER_TILE, ROWS_PER_TILE)],
                    hist_sh.at[pl.ds(sid * ROWS_PER_TILE, ROWS_PER_TILE)])
    plsc.subcore_barrier()

    def body(j, carry):
        base = pl.multiple_of(t * EPT + j * 128, 8)
        pltpu.sync_copy(dst_hbm.at[pl.ds(base, 128)], idx_v)
        pltpu.sync_copy(ones_v, hist_sh.at[idx_v], add=True)
        return carry

    lax.fori_loop(0, NCHUNK, body, 0)
    plsc.subcore_barrier()
    pltpu.sync_copy(hist_sh.at[pl.ds(sid * ROWS_PER_TILE, ROWS_PER_TILE)],
                    out_hbm.at[cid, pl.ds(sid * ROWS_PER_TILE, ROWS_PER_TILE)])


@functools.partial(
    pl.kernel,
    out_type=jax.ShapeDtypeStruct((NC, NPAD, D), jnp.float32),
    mesh=_MESH,
    scratch_types=[
        pltpu.VMEM((128,), jnp.int32),
        pltpu.VMEM((128,), jnp.int32),
        pltpu.VMEM((128, D), jnp.float32),
        pltpu.VMEM_SHARED((NPAD, D), jnp.float32),
        pltpu.SemaphoreType.DMA,
    ],
)
def _sc_scatter(y_hbm, src_hbm, dst_hbm, zeros_hbm, out_hbm,
                src_v, dst_v, rows_v, acc_sh, sem):
    cid = lax.axis_index("c")
    sid = lax.axis_index("s")
    t = cid * NS + sid
    r0 = sid * ROWS_PER_TILE

    pltpu.sync_copy(zeros_hbm.at[pl.ds(r0, ROWS_PER_TILE)],
                    acc_sh.at[pl.ds(r0, ROWS_PER_TILE)])
    plsc.subcore_barrier()

    def body(j, carry):
        base = pl.multiple_of(t * EPT + j * 128, 8)
        pltpu.sync_copy(src_hbm.at[pl.ds(base, 128)], src_v)
        pltpu.sync_copy(dst_hbm.at[pl.ds(base, 128)], dst_v)
        pltpu.async_copy(y_hbm.at[src_v], rows_v, sem).wait()
        pltpu.sync_copy(rows_v, acc_sh.at[dst_v], add=True)
        return carry

    lax.fori_loop(0, NCHUNK, body, 0)
    plsc.subcore_barrier()
    pltpu.sync_copy(acc_sh.at[pl.ds(r0, ROWS_PER_TILE)],
                    out_hbm.at[cid, pl.ds(r0, ROWS_PER_TILE)])


@functools.partial(
    pl.kernel,
    out_type=jax.ShapeDtypeStruct((P, 16), jnp.float32),
    mesh=_MESH,
    scratch_types=[
        pltpu.VMEM((128,), jnp.int32),
        pltpu.VMEM((128,), jnp.int32),
        pltpu.VMEM((128, D), jnp.float32),
        pltpu.VMEM((128, D), jnp.float32),
        pltpu.VMEM((128, 16), jnp.float32),
        pltpu.SemaphoreType.DMA,
        pltpu.SemaphoreType.DMA,
    ],
)
def _sc_linkpred(h_hbm, sp_hbm, dp_hbm, out_hbm,
                 spv, dpv, buf_a, buf_b, res_v, sem_a, sem_b):
    cid = lax.axis_index("c")
    sid = lax.axis_index("s")
    t = cid * NS + sid

    def chunk_body(c, carry):
        base = pl.multiple_of(t * PPT + c * 128, 8)
        pltpu.sync_copy(sp_hbm.at[pl.ds(base, 128)], spv)
        pltpu.sync_copy(dp_hbm.at[pl.ds(base, 128)], dpv)
        cp_a = pltpu.async_copy(h_hbm.at[spv], buf_a, sem_a)
        cp_b = pltpu.async_copy(h_hbm.at[dpv], buf_b, sem_b)
        cp_a.wait()
        cp_b.wait()

        def pair_body(p, carry2):
            prods = [buf_a[p, pl.ds(k * 16, 16)] * buf_b[p, pl.ds(k * 16, 16)]
                     for k in range(8)]
            s01 = prods[0] + prods[1]
            s23 = prods[2] + prods[3]
            s45 = prods[4] + prods[5]
            s67 = prods[6] + prods[7]
            res_v[p, :] = (s01 + s23) + (s45 + s67)
            return carry2

        lax.fori_loop(0, 128, pair_body, 0)
        pltpu.sync_copy(res_v, out_hbm.at[pl.ds(base, 128)])
        return carry

    lax.fori_loop(0, PCHUNK, chunk_body, 0)


BR = 1280
GRID = NPAD // BR


def _tc1_body(x_ref, w_ref, deg_ref, y_ref, dinv_ref):
    deg = deg_ref[...]
    d = (deg[0] + deg[1] + 1).astype(jnp.float32)
    dinv = lax.rsqrt(d)
    y_ref[...] = jnp.dot(x_ref[...], w_ref[...],
                         preferred_element_type=jnp.float32) * dinv
    dinv_ref[...] = dinv


def _tc2_body(s_ref, y_ref, dinv_ref, b_ref, w_ref, y2_ref):
    s = s_ref[...]
    dinv = dinv_ref[...]
    h = jnp.maximum(dinv * (s[0] + s[1] + y_ref[...]) + b_ref[...], 0.0)
    y2_ref[...] = jnp.dot(h, w_ref[...],
                          preferred_element_type=jnp.float32) * dinv


def _tc3_body(s_ref, y_ref, dinv_ref, b_ref, h_ref):
    s = s_ref[...]
    h_ref[...] = dinv_ref[...] * (s[0] + s[1] + y_ref[...]) + b_ref[...]


_row_spec = pl.BlockSpec((BR, D), lambda i: (i, 0))
_w_spec = pl.BlockSpec((D, D), lambda i: (0, 0))
_dinv_spec = pl.BlockSpec((BR, 1), lambda i: (i, 0))
_s_spec = pl.BlockSpec((NC, BR, D), lambda i: (0, i, 0))
_b_spec = pl.BlockSpec((1, D), lambda i: (0, 0))

_tc1 = pl.pallas_call(
    _tc1_body,
    grid=(GRID,),
    in_specs=[_row_spec, _w_spec, pl.BlockSpec((NC, BR, 1), lambda i: (0, i, 0))],
    out_specs=[_row_spec, _dinv_spec],
    out_shape=[jax.ShapeDtypeStruct((NPAD, D), jnp.float32),
               jax.ShapeDtypeStruct((NPAD, 1), jnp.float32)],
)

_tc2 = pl.pallas_call(
    _tc2_body,
    grid=(GRID,),
    in_specs=[_s_spec, _row_spec, _dinv_spec, _b_spec, _w_spec],
    out_specs=_row_spec,
    out_shape=jax.ShapeDtypeStruct((NPAD, D), jnp.float32),
)

_tc3 = pl.pallas_call(
    _tc3_body,
    grid=(GRID,),
    in_specs=[_s_spec, _row_spec, _dinv_spec, _b_spec],
    out_specs=_row_spec,
    out_shape=jax.ShapeDtypeStruct((NPAD, D), jnp.float32),
)


def _tc4_body(r_ref, o_ref):
    z = jnp.sum(r_ref[...], axis=1, keepdims=True)
    o_ref[...] = 1.0 / (1.0 + jnp.exp(-z))


PBR = 8192

_tc4 = pl.pallas_call(
    _tc4_body,
    grid=(P // PBR,),
    in_specs=[pl.BlockSpec((PBR, 16), lambda i: (i, 0))],
    out_specs=pl.BlockSpec((PBR, 1), lambda i: (i, 0)),
    out_shape=jax.ShapeDtypeStruct((P, 1), jnp.float32),
)


@jax.jit
def kernel(x, edge_index, edge_pair, W1, b1, W2, b2):
    src = edge_index[0]
    dst = edge_index[1]
    pad = jnp.full((EPAD - E,), NPAD - 1, jnp.int32)
    srcp = jnp.concatenate([src, pad])
    dstp = jnp.concatenate([dst, pad])
    xp = jnp.pad(x, ((0, NPAD - N), (0, 0)))
    zeros_i = jnp.zeros((NPAD,), jnp.int32)
    zeros_f = jnp.zeros((NPAD, D), jnp.float32)
    b1r = b1.reshape(1, D)
    b2r = b2.reshape(1, D)

    degp = _sc_degree(dstp, zeros_i)
    y1, dinv = _tc1(xp, W1, degp.reshape(NC, NPAD, 1))
    s1 = _sc_scatter(y1, srcp, dstp, zeros_f)
    y2 = _tc2(s1, y1, dinv, b1r, W2)
    s2 = _sc_scatter(y2, srcp, dstp, zeros_f)
    h2 = _tc3(s2, y2, dinv, b2r)
    dots = _sc_linkpred(h2, edge_pair[0], edge_pair[1])
    prob = _tc4(dots)
    return prob.reshape(P)


# speedup vs baseline: 10.2197x; 1.0288x over previous
import functools

import jax
import jax.numpy as jnp
from jax import lax
from jax.experimental import pallas as pl
from jax.experimental.pallas import tpu as pltpu
from jax.experimental.pallas import tpu_sc as plsc

N = 10000
NPAD = 10240
D = 128
E = 320000
NCHUNK = 80
EPT = NCHUNK * 128
EPAD = 32 * EPT
P = 65536
PPT = P // 32
PCHUNK = PPT // 128

NC = 2
NS = 16
ROWS_PER_TILE = NPAD // NS

_MESH = plsc.VectorSubcoreMesh(core_axis_name="c", subcore_axis_name="s")


@functools.partial(
    pl.kernel,
    out_type=jax.ShapeDtypeStruct((NC, NPAD), jnp.int32),
    mesh=_MESH,
    scratch_types=[
        pltpu.VMEM((NCHUNK, 128), jnp.int32),
        pltpu.VMEM((128,), jnp.int32),
        pltpu.VMEM_SHARED((NPAD,), jnp.int32),
    ] + [pltpu.SemaphoreType.DMA] * 8,
)
def _sc_degree(dst_hbm, zeros_hbm, out_hbm, dstI, ones_v, hist_sh, *sems):
    cid = lax.axis_index("c")
    sid = lax.axis_index("s")
    t = cid * NS + sid

    for k in range(8):
        ones_v[pl.ds(k * 16, 16)] = jnp.ones((16,), jnp.int32)
    pltpu.sync_copy(zeros_hbm.at[pl.ds(sid * ROWS_PER_TILE, ROWS_PER_TILE)],
                    hist_sh.at[pl.ds(sid * ROWS_PER_TILE, ROWS_PER_TILE)])
    pltpu.sync_copy(dst_hbm.at[t], dstI)
    plsc.subcore_barrier()

    def body(g, carry):
        for b in range(8):
            pltpu.async_copy(ones_v, hist_sh.at[dstI.at[8 * g + b]],
                             sems[b], add=True)
        for b in range(8):
            pltpu.make_async_copy(ones_v, hist_sh.at[dstI.at[8 * g + b]],
                                  sems[b]).wait()
        return carry

    lax.fori_loop(0, NCHUNK // 8, body, 0)
    plsc.subcore_barrier()
    pltpu.sync_copy(hist_sh.at[pl.ds(sid * ROWS_PER_TILE, ROWS_PER_TILE)],
                    out_hbm.at[cid, pl.ds(sid * ROWS_PER_TILE, ROWS_PER_TILE)])


BLK = 16
NBLK = NCHUNK // BLK


@functools.partial(
    pl.kernel,
    out_type=jax.ShapeDtypeStruct((NC, NPAD, D), jnp.float32),
    mesh=_MESH,
    scratch_types=[
        pltpu.VMEM((2, BLK, 128), jnp.int32),
        pltpu.VMEM((2, BLK, 128), jnp.int32),
        pltpu.VMEM((2, 128, D), jnp.float32),
        pltpu.VMEM_SHARED((NPAD, D), jnp.float32),
    ] + [pltpu.SemaphoreType.DMA] * 4,
)
def _sc_scatter(y_hbm, src_hbm, dst_hbm, zeros_hbm, out_hbm,
                srcB, dstB, rows, acc_sh, *sems):
    cid = lax.axis_index("c")
    sid = lax.axis_index("s")
    t = cid * NS + sid
    r0 = sid * ROWS_PER_TILE

    pltpu.sync_copy(zeros_hbm.at[pl.ds(r0, ROWS_PER_TILE)],
                    acc_sh.at[pl.ds(r0, ROWS_PER_TILE)])
    pltpu.async_copy(src_hbm.at[t, pl.ds(0, BLK)], srcB.at[0], sems[2])
    pltpu.async_copy(dst_hbm.at[t, pl.ds(0, BLK)], dstB.at[0], sems[3])
    plsc.subcore_barrier()

    for blk in range(NBLK):
        sb = blk % 2
        pltpu.make_async_copy(src_hbm.at[t, pl.ds(0, BLK)], srcB.at[sb],
                              sems[2]).wait()
        pltpu.make_async_copy(dst_hbm.at[t, pl.ds(0, BLK)], dstB.at[sb],
                              sems[3]).wait()
        if blk + 1 < NBLK:
            pltpu.async_copy(src_hbm.at[t, pl.ds((blk + 1) * BLK, BLK)],
                             srcB.at[1 - sb], sems[2])
            pltpu.async_copy(dst_hbm.at[t, pl.ds((blk + 1) * BLK, BLK)],
                             dstB.at[1 - sb], sems[3])
        for b in range(2):
            pltpu.async_copy(y_hbm.at[srcB.at[sb, b]], rows.at[b], sems[b])

        def body(g, carry, sb=sb):
            for b in range(2):
                lj = 2 * g + b
                pltpu.make_async_copy(y_hbm.at[srcB.at[sb, lj]], rows.at[b],
                                      sems[b]).wait()
                pltpu.sync_copy(rows.at[b], acc_sh.at[dstB.at[sb, lj]],
                                add=True)

                @pl.when(lj < BLK - 2)
                def _():
                    pltpu.async_copy(y_hbm.at[srcB.at[sb, lj + 2]],
                                     rows.at[b], sems[b])
            return carry

        lax.fori_loop(0, BLK // 2, body, 0)

    plsc.subcore_barrier()
    pltpu.sync_copy(acc_sh.at[pl.ds(r0, ROWS_PER_TILE)],
                    out_hbm.at[cid, pl.ds(r0, ROWS_PER_TILE)])


@functools.partial(
    pl.kernel,
    out_type=jax.ShapeDtypeStruct((P, 16), jnp.float32),
    mesh=_MESH,
    scratch_types=[
        pltpu.VMEM((PCHUNK, 128), jnp.int32),
        pltpu.VMEM((PCHUNK, 128), jnp.int32),
        pltpu.VMEM((2, 128, D), jnp.float32),
        pltpu.VMEM((2, 128, D), jnp.float32),
        pltpu.VMEM((128, 16), jnp.float32),
    ] + [pltpu.SemaphoreType.DMA] * 4,
)
def _sc_linkpred(h_hbm, sp_hbm, dp_hbm, out_hbm,
                 spI, dpI, buf_a, buf_b, res_v, *sems):
    cid = lax.axis_index("c")
    sid = lax.axis_index("s")
    t = cid * NS + sid

    pltpu.sync_copy(sp_hbm.at[t], spI)
    pltpu.sync_copy(dp_hbm.at[t], dpI)
    for s in range(2):
        pltpu.async_copy(h_hbm.at[spI.at[s]], buf_a.at[s], sems[2 * s])
        pltpu.async_copy(h_hbm.at[dpI.at[s]], buf_b.at[s], sems[2 * s + 1])

    def chunk_pair_body(g, carry):
        for s in range(2):
            c = 2 * g + s
            pltpu.make_async_copy(h_hbm.at[spI.at[c]], buf_a.at[s],
                                  sems[2 * s]).wait()
            pltpu.make_async_copy(h_hbm.at[dpI.at[c]], buf_b.at[s],
                                  sems[2 * s + 1]).wait()

            def pair_body(p, carry2):
                prods = [buf_a[s, p, pl.ds(k * 16, 16)]
                         * buf_b[s, p, pl.ds(k * 16, 16)] for k in range(8)]
                s01 = prods[0] + prods[1]
                s23 = prods[2] + prods[3]
                s45 = prods[4] + prods[5]
                s67 = prods[6] + prods[7]
                res_v[p, :] = (s01 + s23) + (s45 + s67)
                return carry2

            lax.fori_loop(0, 128, pair_body, 0)
            base = pl.multiple_of(t * PPT + c * 128, 8)
            pltpu.sync_copy(res_v, out_hbm.at[pl.ds(base, 128)])

            @pl.when(g < PCHUNK // 2 - 1)
            def _():
                pltpu.async_copy(h_hbm.at[spI.at[c + 2]], buf_a.at[s],
                                 sems[2 * s])
                pltpu.async_copy(h_hbm.at[dpI.at[c + 2]], buf_b.at[s],
                                 sems[2 * s + 1])
        return carry

    lax.fori_loop(0, PCHUNK // 2, chunk_pair_body, 0)


BR = 1280
GRID = NPAD // BR


def _tc1_body(x_ref, w_ref, deg_ref, y_ref, dinv_ref):
    deg = deg_ref[...]
    d = (deg[0] + deg[1] + 1).astype(jnp.float32)
    dinv = lax.rsqrt(d)
    y_ref[...] = jnp.dot(x_ref[...], w_ref[...],
                         preferred_element_type=jnp.float32) * dinv
    dinv_ref[...] = dinv


def _tc2_body(s_ref, y_ref, dinv_ref, b_ref, w_ref, y2_ref):
    s = s_ref[...]
    dinv = dinv_ref[...]
    h = jnp.maximum(dinv * (s[0] + s[1] + y_ref[...]) + b_ref[...], 0.0)
    y2_ref[...] = jnp.dot(h, w_ref[...],
                          preferred_element_type=jnp.float32) * dinv


def _tc3_body(s_ref, y_ref, dinv_ref, b_ref, h_ref):
    s = s_ref[...]
    h_ref[...] = dinv_ref[...] * (s[0] + s[1] + y_ref[...]) + b_ref[...]


_row_spec = pl.BlockSpec((BR, D), lambda i: (i, 0))
_w_spec = pl.BlockSpec((D, D), lambda i: (0, 0))
_dinv_spec = pl.BlockSpec((BR, 1), lambda i: (i, 0))
_s_spec = pl.BlockSpec((NC, BR, D), lambda i: (0, i, 0))
_b_spec = pl.BlockSpec((1, D), lambda i: (0, 0))

_tc1 = pl.pallas_call(
    _tc1_body,
    grid=(GRID,),
    in_specs=[_row_spec, _w_spec, pl.BlockSpec((NC, BR, 1), lambda i: (0, i, 0))],
    out_specs=[_row_spec, _dinv_spec],
    out_shape=[jax.ShapeDtypeStruct((NPAD, D), jnp.float32),
               jax.ShapeDtypeStruct((NPAD, 1), jnp.float32)],
)

_tc2 = pl.pallas_call(
    _tc2_body,
    grid=(GRID,),
    in_specs=[_s_spec, _row_spec, _dinv_spec, _b_spec, _w_spec],
    out_specs=_row_spec,
    out_shape=jax.ShapeDtypeStruct((NPAD, D), jnp.float32),
)

_tc3 = pl.pallas_call(
    _tc3_body,
    grid=(GRID,),
    in_specs=[_s_spec, _row_spec, _dinv_spec, _b_spec],
    out_specs=_row_spec,
    out_shape=jax.ShapeDtypeStruct((NPAD, D), jnp.float32),
)


def _tc4_body(r_ref, o_ref):
    z = jnp.sum(r_ref[...], axis=1, keepdims=True)
    o_ref[...] = 1.0 / (1.0 + jnp.exp(-z))


PBR = 8192

_tc4 = pl.pallas_call(
    _tc4_body,
    grid=(P // PBR,),
    in_specs=[pl.BlockSpec((PBR, 16), lambda i: (i, 0))],
    out_specs=pl.BlockSpec((PBR, 1), lambda i: (i, 0)),
    out_shape=jax.ShapeDtypeStruct((P, 1), jnp.float32),
)


@jax.jit
def kernel(x, edge_index, edge_pair, W1, b1, W2, b2):
    src = edge_index[0]
    dst = edge_index[1]
    pad = jnp.full((EPAD - E,), NPAD - 1, jnp.int32)
    srcp = jnp.concatenate([src, pad]).reshape(32, NCHUNK, 128)
    dstp = jnp.concatenate([dst, pad]).reshape(32, NCHUNK, 128)
    xp = jnp.pad(x, ((0, NPAD - N), (0, 0)))
    zeros_i = jnp.zeros((NPAD,), jnp.int32)
    zeros_f = jnp.zeros((NPAD, D), jnp.float32)
    b1r = b1.reshape(1, D)
    b2r = b2.reshape(1, D)
    sp3 = edge_pair[0].reshape(32, PCHUNK, 128)
    dp3 = edge_pair[1].reshape(32, PCHUNK, 128)

    degp = _sc_degree(dstp, zeros_i)
    y1, dinv = _tc1(xp, W1, degp.reshape(NC, NPAD, 1))
    s1 = _sc_scatter(y1, srcp, dstp, zeros_f)
    y2 = _tc2(s1, y1, dinv, b1r, W2)
    s2 = _sc_scatter(y2, srcp, dstp, zeros_f)
    h2 = _tc3(s2, y2, dinv, b2r)
    dots = _sc_linkpred(h2, sp3, dp3)
    prob = _tc4(dots)
    return prob.reshape(P)


# speedup vs baseline: 26.0593x; 2.5499x over previous
import functools

import jax
import jax.numpy as jnp
from jax import lax
from jax.experimental import pallas as pl
from jax.experimental.pallas import tpu as pltpu
from jax.experimental.pallas import tpu_sc as plsc

N = 10000
NPAD = 10240
D = 128
E = 320000
NCHUNK = 80
EPT = NCHUNK * 128
EPAD = 32 * EPT
P = 65536
PPT = P // 32
PCHUNK = PPT // 128

NC = 2
NS = 16
ROWS_PER_TILE = NPAD // NS

_MESH = plsc.VectorSubcoreMesh(core_axis_name="c", subcore_axis_name="s")


@functools.partial(
    pl.kernel,
    out_type=jax.ShapeDtypeStruct((NC, NPAD), jnp.int32),
    mesh=_MESH,
    scratch_types=[
        pltpu.VMEM((NCHUNK, 128), jnp.int32),
        pltpu.VMEM((128,), jnp.int32),
        pltpu.VMEM_SHARED((NPAD,), jnp.int32),
    ] + [pltpu.SemaphoreType.DMA] * 8,
)
def _sc_degree(dst_hbm, zeros_hbm, out_hbm, dstI, ones_v, hist_sh, *sems):
    cid = lax.axis_index("c")
    sid = lax.axis_index("s")
    t = cid * NS + sid

    for k in range(8):
        ones_v[pl.ds(k * 16, 16)] = jnp.ones((16,), jnp.int32)
    pltpu.sync_copy(zeros_hbm.at[pl.ds(sid * ROWS_PER_TILE, ROWS_PER_TILE)],
                    hist_sh.at[pl.ds(sid * ROWS_PER_TILE, ROWS_PER_TILE)])
    pltpu.sync_copy(dst_hbm.at[t], dstI)
    plsc.subcore_barrier()

    def body(g, carry):
        for b in range(8):
            pltpu.async_copy(ones_v, hist_sh.at[dstI.at[8 * g + b]],
                             sems[b], add=True)
        for b in range(8):
            pltpu.make_async_copy(ones_v, hist_sh.at[dstI.at[8 * g + b]],
                                  sems[b]).wait()
        return carry

    lax.fori_loop(0, NCHUNK // 8, body, 0)
    plsc.subcore_barrier()
    pltpu.sync_copy(hist_sh.at[pl.ds(sid * ROWS_PER_TILE, ROWS_PER_TILE)],
                    out_hbm.at[cid, pl.ds(sid * ROWS_PER_TILE, ROWS_PER_TILE)])


BLK = 16
NBLK = NCHUNK // BLK


@functools.partial(
    pl.kernel,
    out_type=jax.ShapeDtypeStruct((NC, NPAD, D), jnp.float32),
    mesh=_MESH,
    scratch_types=[
        pltpu.VMEM((2, BLK, 128), jnp.int32),
        pltpu.VMEM((2, BLK, 128), jnp.int32),
        pltpu.VMEM((2, 128, D), jnp.float32),
        pltpu.VMEM_SHARED((NPAD, D), jnp.float32),
    ] + [pltpu.SemaphoreType.DMA] * 4,
)
def _sc_scatter(y_hbm, src_hbm, dst_hbm, zeros_hbm, out_hbm,
                srcB, dstB, rows, acc_sh, *sems):
    cid = lax.axis_index("c")
    sid = lax.axis_index("s")
    t = cid * NS + sid
    r0 = sid * ROWS_PER_TILE

    pltpu.sync_copy(zeros_hbm.at[pl.ds(r0, ROWS_PER_TILE)],
                    acc_sh.at[pl.ds(r0, ROWS_PER_TILE)])
    pltpu.async_copy(src_hbm.at[t, pl.ds(0, BLK)], srcB.at[0], sems[2])
    pltpu.async_copy(dst_hbm.at[t, pl.ds(0, BLK)], dstB.at[0], sems[3])
    plsc.subcore_barrier()

    for blk in range(NBLK):
        sb = blk % 2
        pltpu.make_async_copy(src_hbm.at[t, pl.ds(0, BLK)], srcB.at[sb],
                              sems[2]).wait()
        pltpu.make_async_copy(dst_hbm.at[t, pl.ds(0, BLK)], dstB.at[sb],
                              sems[3]).wait()
        if blk + 1 < NBLK:
            pltpu.async_copy(src_hbm.at[t, pl.ds((blk + 1) * BLK, BLK)],
                             srcB.at[1 - sb], sems[2])
            pltpu.async_copy(dst_hbm.at[t, pl.ds((blk + 1) * BLK, BLK)],
                             dstB.at[1 - sb], sems[3])
        for b in range(2):
            pltpu.async_copy(y_hbm.at[srcB.at[sb, b]], rows.at[b], sems[b])

        def body(g, carry, sb=sb):
            for b in range(2):
                lj = 2 * g + b
                pltpu.make_async_copy(y_hbm.at[srcB.at[sb, lj]], rows.at[b],
                                      sems[b]).wait()
                pltpu.sync_copy(rows.at[b], acc_sh.at[dstB.at[sb, lj]],
                                add=True)

                @pl.when(lj < BLK - 2)
                def _():
                    pltpu.async_copy(y_hbm.at[srcB.at[sb, lj + 2]],
                                     rows.at[b], sems[b])
            return carry

        lax.fori_loop(0, BLK // 2, body, 0)

    plsc.subcore_barrier()
    pltpu.sync_copy(acc_sh.at[pl.ds(r0, ROWS_PER_TILE)],
                    out_hbm.at[cid, pl.ds(r0, ROWS_PER_TILE)])


@functools.partial(
    pl.kernel,
    out_type=jax.ShapeDtypeStruct((P, 16), jnp.float32),
    mesh=_MESH,
    scratch_types=[
        pltpu.VMEM((PCHUNK, 128), jnp.int32),
        pltpu.VMEM((PCHUNK, 128), jnp.int32),
        pltpu.VMEM((2, 128, D), jnp.float32),
        pltpu.VMEM((2, 128, D), jnp.float32),
        pltpu.VMEM((128, 16), jnp.float32),
    ] + [pltpu.SemaphoreType.DMA] * 4,
)
def _sc_linkpred(h_hbm, sp_hbm, dp_hbm, out_hbm,
                 spI, dpI, buf_a, buf_b, res_v, *sems):
    cid = lax.axis_index("c")
    sid = lax.axis_index("s")
    t = cid * NS + sid

    pltpu.sync_copy(sp_hbm.at[t], spI)
    pltpu.sync_copy(dp_hbm.at[t], dpI)
    for s in range(2):
        pltpu.async_copy(h_hbm.at[spI.at[s]], buf_a.at[s], sems[2 * s])
        pltpu.async_copy(h_hbm.at[dpI.at[s]], buf_b.at[s], sems[2 * s + 1])

    def chunk_pair_body(g, carry):
        for s in range(2):
            c = 2 * g + s
            pltpu.make_async_copy(h_hbm.at[spI.at[c]], buf_a.at[s],
                                  sems[2 * s]).wait()
            pltpu.make_async_copy(h_hbm.at[dpI.at[c]], buf_b.at[s],
                                  sems[2 * s + 1]).wait()

            def pair_body(p, carry2):
                prods = [buf_a[s, p, pl.ds(k * 16, 16)]
                         * buf_b[s, p, pl.ds(k * 16, 16)] for k in range(8)]
                s01 = prods[0] + prods[1]
                s23 = prods[2] + prods[3]
                s45 = prods[4] + prods[5]
                s67 = prods[6] + prods[7]
                res_v[p, :] = (s01 + s23) + (s45 + s67)
                return carry2

            lax.fori_loop(0, 128, pair_body, 0)
            base = pl.multiple_of(t * PPT + c * 128, 8)
            pltpu.sync_copy(res_v, out_hbm.at[pl.ds(base, 128)])

            @pl.when(g < PCHUNK // 2 - 1)
            def _():
                pltpu.async_copy(h_hbm.at[spI.at[c + 2]], buf_a.at[s],
                                 sems[2 * s])
                pltpu.async_copy(h_hbm.at[dpI.at[c + 2]], buf_b.at[s],
                                 sems[2 * s + 1])
        return carry

    lax.fori_loop(0, PCHUNK // 2, chunk_pair_body, 0)


BR = 1280
GRID = NPAD // BR


def _tc1_body(x_ref, w_ref, deg_ref, y_ref, dinv_ref):
    deg = deg_ref[...]
    d = (deg[0] + deg[1] + 1).astype(jnp.float32)
    dinv = lax.rsqrt(d)
    y_ref[...] = jnp.dot(x_ref[...], w_ref[...],
                         preferred_element_type=jnp.float32) * dinv
    dinv_ref[...] = dinv


def _tc2_body(s_ref, y_ref, dinv_ref, b_ref, w_ref, y2_ref):
    s = s_ref[...]
    dinv = dinv_ref[...]
    h = jnp.maximum(dinv * (s[0] + s[1] + y_ref[...]) + b_ref[...], 0.0)
    y2_ref[...] = jnp.dot(h, w_ref[...],
                          preferred_element_type=jnp.float32) * dinv


def _tc3_body(s_ref, y_ref, dinv_ref, b_ref, h_ref):
    s = s_ref[...]
    h_ref[...] = dinv_ref[...] * (s[0] + s[1] + y_ref[...]) + b_ref[...]


_row_spec = pl.BlockSpec((BR, D), lambda i: (i, 0))
_w_spec = pl.BlockSpec((D, D), lambda i: (0, 0))
_dinv_spec = pl.BlockSpec((BR, 1), lambda i: (i, 0))
_s_spec = pl.BlockSpec((NC, BR, D), lambda i: (0, i, 0))
_b_spec = pl.BlockSpec((1, D), lambda i: (0, 0))

_tc1 = pl.pallas_call(
    _tc1_body,
    grid=(GRID,),
    in_specs=[_row_spec, _w_spec, pl.BlockSpec((NC, BR, 1), lambda i: (0, i, 0))],
    out_specs=[_row_spec, _dinv_spec],
    out_shape=[jax.ShapeDtypeStruct((NPAD, D), jnp.float32),
               jax.ShapeDtypeStruct((NPAD, 1), jnp.float32)],
)

_tc2 = pl.pallas_call(
    _tc2_body,
    grid=(GRID,),
    in_specs=[_s_spec, _row_spec, _dinv_spec, _b_spec, _w_spec],
    out_specs=_row_spec,
    out_shape=jax.ShapeDtypeStruct((NPAD, D), jnp.float32),
)

_tc3 = pl.pallas_call(
    _tc3_body,
    grid=(GRID,),
    in_specs=[_s_spec, _row_spec, _dinv_spec, _b_spec],
    out_specs=_row_spec,
    out_shape=jax.ShapeDtypeStruct((NPAD, D), jnp.float32),
)


def _tc4_body(r_ref, o_ref):
    z = jnp.sum(r_ref[...], axis=1, keepdims=True)
    o_ref[...] = 1.0 / (1.0 + jnp.exp(-z))


PBR = 8192

_tc4 = pl.pallas_call(
    _tc4_body,
    grid=(P // PBR,),
    in_specs=[pl.BlockSpec((PBR, 16), lambda i: (i, 0))],
    out_specs=pl.BlockSpec((PBR, 1), lambda i: (i, 0)),
    out_shape=jax.ShapeDtypeStruct((P, 1), jnp.float32),
)


@jax.jit
def kernel(x, edge_index, edge_pair, W1, b1, W2, b2):
    src = edge_index[0]
    dst = edge_index[1]
    pad = N + jnp.arange(EPAD - E, dtype=jnp.int32) % (NPAD - N)
    srcp = jnp.concatenate([src, pad]).reshape(32, NCHUNK, 128)
    dstp = jnp.concatenate([dst, pad]).reshape(32, NCHUNK, 128)
    xp = jnp.pad(x, ((0, NPAD - N), (0, 0)))
    zeros_i = jnp.zeros((NPAD,), jnp.int32)
    zeros_f = jnp.zeros((NPAD, D), jnp.float32)
    b1r = b1.reshape(1, D)
    b2r = b2.reshape(1, D)
    sp3 = edge_pair[0].reshape(32, PCHUNK, 128)
    dp3 = edge_pair[1].reshape(32, PCHUNK, 128)

    degp = _sc_degree(dstp, zeros_i)
    y1, dinv = _tc1(xp, W1, degp.reshape(NC, NPAD, 1))
    s1 = _sc_scatter(y1, srcp, dstp, zeros_f)
    y2 = _tc2(s1, y1, dinv, b1r, W2)
    s2 = _sc_scatter(y2, srcp, dstp, zeros_f)
    h2 = _tc3(s2, y2, dinv, b2r)
    dots = _sc_linkpred(h2, sp3, dp3)
    prob = _tc4(dots)
    return prob.reshape(P)


# speedup vs baseline: 26.1220x; 1.0024x over previous
import functools

import jax
import jax.numpy as jnp
from jax import lax
from jax.experimental import pallas as pl
from jax.experimental.pallas import tpu as pltpu
from jax.experimental.pallas import tpu_sc as plsc

N = 10000
NPAD = 10240
D = 128
E = 320000
EPT = 10368
EPAD = 32 * EPT
HCHUNK = 81
C = 96
SCHUNK = EPT // C
SLOTS = 3
BLKC = 12
NBLKS = SCHUNK // BLKC
P = 65536
PPT = P // 32
PCHUNK = PPT // 128

NC = 2
NS = 16
ROWS_PER_TILE = NPAD // NS

_MESH = plsc.VectorSubcoreMesh(core_axis_name="c", subcore_axis_name="s")


@functools.partial(
    pl.kernel,
    out_type=jax.ShapeDtypeStruct((NC, NPAD), jnp.int32),
    mesh=_MESH,
    scratch_types=[
        pltpu.VMEM((HCHUNK, 128), jnp.int32),
        pltpu.VMEM((128,), jnp.int32),
        pltpu.VMEM_SHARED((NPAD,), jnp.int32),
    ] + [pltpu.SemaphoreType.DMA] * 9,
)
def _sc_degree(dst_hbm, zeros_hbm, out_hbm, dstI, ones_v, hist_sh, *sems):
    cid = lax.axis_index("c")
    sid = lax.axis_index("s")
    t = cid * NS + sid

    for k in range(8):
        ones_v[pl.ds(k * 16, 16)] = jnp.ones((16,), jnp.int32)
    pltpu.sync_copy(zeros_hbm.at[pl.ds(sid * ROWS_PER_TILE, ROWS_PER_TILE)],
                    hist_sh.at[pl.ds(sid * ROWS_PER_TILE, ROWS_PER_TILE)])
    pltpu.sync_copy(dst_hbm.at[t], dstI)
    plsc.subcore_barrier()

    def body(g, carry):
        for b in range(9):
            pltpu.async_copy(ones_v, hist_sh.at[dstI.at[9 * g + b]],
                             sems[b], add=True)
        for b in range(9):
            pltpu.make_async_copy(ones_v, hist_sh.at[dstI.at[9 * g + b]],
                                  sems[b]).wait()
        return carry

    lax.fori_loop(0, HCHUNK // 9, body, 0)
    plsc.subcore_barrier()
    pltpu.sync_copy(hist_sh.at[pl.ds(sid * ROWS_PER_TILE, ROWS_PER_TILE)],
                    out_hbm.at[cid, pl.ds(sid * ROWS_PER_TILE, ROWS_PER_TILE)])


@functools.partial(
    pl.kernel,
    out_type=jax.ShapeDtypeStruct((NC, NPAD, D), jnp.float32),
    mesh=_MESH,
    scratch_types=[
        pltpu.VMEM((2, BLKC, C), jnp.int32),
        pltpu.VMEM((2, BLKC, C), jnp.int32),
        pltpu.VMEM((SLOTS, C, D), jnp.float32),
        pltpu.VMEM_SHARED((NPAD, D), jnp.float32),
    ] + [pltpu.SemaphoreType.DMA] * (SLOTS + 2),
)
def _sc_scatter(y_hbm, src_hbm, dst_hbm, zeros_hbm, out_hbm,
                srcB, dstB, rows, acc_sh, *sems):
    cid = lax.axis_index("c")
    sid = lax.axis_index("s")
    t = cid * NS + sid
    r0 = sid * ROWS_PER_TILE

    pltpu.sync_copy(zeros_hbm.at[pl.ds(r0, ROWS_PER_TILE)],
                    acc_sh.at[pl.ds(r0, ROWS_PER_TILE)])
    pltpu.async_copy(src_hbm.at[t, 0], srcB.at[0], sems[SLOTS])
    pltpu.async_copy(dst_hbm.at[t, 0], dstB.at[0], sems[SLOTS + 1])
    plsc.subcore_barrier()

    for blk in range(NBLKS):
        sb = blk % 2
        pltpu.make_async_copy(src_hbm.at[t, 0], srcB.at[sb],
                              sems[SLOTS]).wait()
        pltpu.make_async_copy(dst_hbm.at[t, 0], dstB.at[sb],
                              sems[SLOTS + 1]).wait()
        if blk + 1 < NBLKS:
            pltpu.async_copy(src_hbm.at[t, blk + 1], srcB.at[1 - sb],
                             sems[SLOTS])
            pltpu.async_copy(dst_hbm.at[t, blk + 1], dstB.at[1 - sb],
                             sems[SLOTS + 1])
        for b in range(SLOTS):
            pltpu.async_copy(y_hbm.at[srcB.at[sb, b]], rows.at[b], sems[b])

        def body(g, carry, sb=sb):
            for b in range(SLOTS):
                lj = SLOTS * g + b
                pltpu.make_async_copy(y_hbm.at[srcB.at[sb, lj]], rows.at[b],
                                      sems[b]).wait()
                pltpu.sync_copy(rows.at[b], acc_sh.at[dstB.at[sb, lj]],
                                add=True)

                @pl.when(lj < BLKC - SLOTS)
                def _():
                    pltpu.async_copy(y_hbm.at[srcB.at[sb, lj + SLOTS]],
                                     rows.at[b], sems[b])
            return carry

        lax.fori_loop(0, BLKC // SLOTS, body, 0)

    plsc.subcore_barrier()
    pltpu.sync_copy(acc_sh.at[pl.ds(r0, ROWS_PER_TILE)],
                    out_hbm.at[cid, pl.ds(r0, ROWS_PER_TILE)])


@functools.partial(
    pl.kernel,
    out_type=jax.ShapeDtypeStruct((P, 16), jnp.float32),
    mesh=_MESH,
    scratch_types=[
        pltpu.VMEM((PCHUNK, 128), jnp.int32),
        pltpu.VMEM((PCHUNK, 128), jnp.int32),
        pltpu.VMEM((2, 128, D), jnp.float32),
        pltpu.VMEM((2, 128, D), jnp.float32),
        pltpu.VMEM((128, 16), jnp.float32),
    ] + [pltpu.SemaphoreType.DMA] * 4,
)
def _sc_linkpred(h_hbm, sp_hbm, dp_hbm, out_hbm,
                 spI, dpI, buf_a, buf_b, res_v, *sems):
    cid = lax.axis_index("c")
    sid = lax.axis_index("s")
    t = cid * NS + sid

    pltpu.sync_copy(sp_hbm.at[t], spI)
    pltpu.sync_copy(dp_hbm.at[t], dpI)
    for s in range(2):
        pltpu.async_copy(h_hbm.at[spI.at[s]], buf_a.at[s], sems[2 * s])
        pltpu.async_copy(h_hbm.at[dpI.at[s]], buf_b.at[s], sems[2 * s + 1])

    def chunk_pair_body(g, carry):
        for s in range(2):
            c = 2 * g + s
            pltpu.make_async_copy(h_hbm.at[spI.at[c]], buf_a.at[s],
                                  sems[2 * s]).wait()
            pltpu.make_async_copy(h_hbm.at[dpI.at[c]], buf_b.at[s],
                                  sems[2 * s + 1]).wait()

            def pair_body(p, carry2):
                prods = [buf_a[s, p, pl.ds(k * 16, 16)]
                         * buf_b[s, p, pl.ds(k * 16, 16)] for k in range(8)]
                s01 = prods[0] + prods[1]
                s23 = prods[2] + prods[3]
                s45 = prods[4] + prods[5]
                s67 = prods[6] + prods[7]
                res_v[p, :] = (s01 + s23) + (s45 + s67)
                return carry2

            lax.fori_loop(0, 128, pair_body, 0)
            base = pl.multiple_of(t * PPT + c * 128, 8)
            pltpu.sync_copy(res_v, out_hbm.at[pl.ds(base, 128)])

            @pl.when(g < PCHUNK // 2 - 1)
            def _():
                pltpu.async_copy(h_hbm.at[spI.at[c + 2]], buf_a.at[s],
                                 sems[2 * s])
                pltpu.async_copy(h_hbm.at[dpI.at[c + 2]], buf_b.at[s],
                                 sems[2 * s + 1])
        return carry

    lax.fori_loop(0, PCHUNK // 2, chunk_pair_body, 0)


BR = 1280
GRID = NPAD // BR


def _tc1_body(x_ref, w_ref, deg_ref, y_ref, dinv_ref):
    deg = deg_ref[...]
    d = (deg[0] + deg[1] + 1).astype(jnp.float32)
    dinv = lax.rsqrt(d)
    y_ref[...] = jnp.dot(x_ref[...], w_ref[...],
                         preferred_element_type=jnp.float32) * dinv
    dinv_ref[...] = dinv


def _tc2_body(s_ref, y_ref, dinv_ref, b_ref, w_ref, y2_ref):
    s = s_ref[...]
    dinv = dinv_ref[...]
    h = jnp.maximum(dinv * (s[0] + s[1] + y_ref[...]) + b_ref[...], 0.0)
    y2_ref[...] = jnp.dot(h, w_ref[...],
                          preferred_element_type=jnp.float32) * dinv


def _tc3_body(s_ref, y_ref, dinv_ref, b_ref, h_ref):
    s = s_ref[...]
    h_ref[...] = dinv_ref[...] * (s[0] + s[1] + y_ref[...]) + b_ref[...]


_row_spec = pl.BlockSpec((BR, D), lambda i: (i, 0))
_w_spec = pl.BlockSpec((D, D), lambda i: (0, 0))
_dinv_spec = pl.BlockSpec((BR, 1), lambda i: (i, 0))
_s_spec = pl.BlockSpec((NC, BR, D), lambda i: (0, i, 0))
_b_spec = pl.BlockSpec((1, D), lambda i: (0, 0))

_tc1 = pl.pallas_call(
    _tc1_body,
    grid=(GRID,),
    in_specs=[_row_spec, _w_spec, pl.BlockSpec((NC, BR, 1), lambda i: (0, i, 0))],
    out_specs=[_row_spec, _dinv_spec],
    out_shape=[jax.ShapeDtypeStruct((NPAD, D), jnp.float32),
               jax.ShapeDtypeStruct((NPAD, 1), jnp.float32)],
)

_tc2 = pl.pallas_call(
    _tc2_body,
    grid=(GRID,),
    in_specs=[_s_spec, _row_spec, _dinv_spec, _b_spec, _w_spec],
    out_specs=_row_spec,
    out_shape=jax.ShapeDtypeStruct((NPAD, D), jnp.float32),
)

_tc3 = pl.pallas_call(
    _tc3_body,
    grid=(GRID,),
    in_specs=[_s_spec, _row_spec, _dinv_spec, _b_spec],
    out_specs=_row_spec,
    out_shape=jax.ShapeDtypeStruct((NPAD, D), jnp.float32),
)


def _tc4_body(r_ref, o_ref):
    z = jnp.sum(r_ref[...], axis=1, keepdims=True)
    o_ref[...] = 1.0 / (1.0 + jnp.exp(-z))


PBR = 8192

_tc4 = pl.pallas_call(
    _tc4_body,
    grid=(P // PBR,),
    in_specs=[pl.BlockSpec((PBR, 16), lambda i: (i, 0))],
    out_specs=pl.BlockSpec((PBR, 1), lambda i: (i, 0)),
    out_shape=jax.ShapeDtypeStruct((P, 1), jnp.float32),
)


@jax.jit
def kernel(x, edge_index, edge_pair, W1, b1, W2, b2):
    src = edge_index[0]
    dst = edge_index[1]
    pad = N + jnp.arange(EPAD - E, dtype=jnp.int32) % (NPAD - N)
    srcp = jnp.concatenate([src, pad]).reshape(32, NBLKS, BLKC, C)
    dstp = jnp.concatenate([dst, pad]).reshape(32, NBLKS, BLKC, C)
    dstp_h = dstp.reshape(32, HCHUNK, 128)
    xp = jnp.pad(x, ((0, NPAD - N), (0, 0)))
    zeros_i = jnp.zeros((NPAD,), jnp.int32)
    zeros_f = jnp.zeros((NPAD, D), jnp.float32)
    b1r = b1.reshape(1, D)
    b2r = b2.reshape(1, D)
    sp3 = edge_pair[0].reshape(32, PCHUNK, 128)
    dp3 = edge_pair[1].reshape(32, PCHUNK, 128)

    degp = _sc_degree(dstp_h, zeros_i)
    y1, dinv = _tc1(xp, W1, degp.reshape(NC, NPAD, 1))
    s1 = _sc_scatter(y1, srcp, dstp, zeros_f)
    y2 = _tc2(s1, y1, dinv, b1r, W2)
    s2 = _sc_scatter(y2, srcp, dstp, zeros_f)
    h2 = _tc3(s2, y2, dinv, b2r)
    dots = _sc_linkpred(h2, sp3, dp3)
    prob = _tc4(dots)
    return prob.reshape(P)
